# baseline (speedup 1.0000x reference)
import functools

import jax
import jax.numpy as jnp
import numpy as np
from jax.experimental import pallas as pl
from jax.experimental.pallas import tpu as pltpu

BLOCK = 128
NB = 16
H = 16
S = 2048
D = 128
SCALE = 1.0 / np.sqrt(D)
MIN_RETAIN = 1
MAX_RETAIN = 11
ENERGY = 0.95
QTILE = 512
NQT = S // QTILE
QB_PER_TILE = QTILE // BLOCK
KCHUNK = 128
NKC = S // KCHUNK
KB_PER_CHUNK = KCHUNK // BLOCK


def _mask_kernel(q_ref, k_ref, cbm_ref, qb_ref, kb_ref):
    q = q_ref[0]
    k = k_ref[0]
    qb_ref[0] = (q * SCALE).astype(jnp.bfloat16)
    kb_ref[0] = k.astype(jnp.bfloat16)

    qp = q.reshape(NB, BLOCK, D).mean(axis=1)
    kp = k.reshape(NB, BLOCK, D).mean(axis=1)
    s = jax.lax.dot_general(qp, kp, (((1,), (1,)), ((), ())),
                            preferred_element_type=jnp.float32) * SCALE
    s = s - s.max(axis=-1, keepdims=True)
    e = jnp.exp(s)
    p = e / e.sum(axis=-1, keepdims=True)

    l_idx = jax.lax.broadcasted_iota(jnp.int32, (NB, NB), 1)
    rank_cols = []
    scum_cols = []
    for j in range(NB):
        pj = p[:, j:j + 1]
        before = (p > pj) | ((p == pj) & (l_idx < j))
        rank_cols.append(before.sum(axis=1, keepdims=True).astype(jnp.int32))
        scum_cols.append(pj + jnp.sum(jnp.where(before, p, 0.0), axis=1,
                                      keepdims=True))
    rank = jnp.concatenate(rank_cols, axis=1)
    scum = jnp.concatenate(scum_cols, axis=1)

    theta = ENERGY * p.sum(axis=-1, keepdims=True)
    k_idx = jnp.sum((scum < theta).astype(jnp.int32), axis=1, keepdims=True)
    k_keep = jnp.clip(k_idx, MIN_RETAIN, MAX_RETAIN)
    keep = rank < k_keep

    qsq = jnp.sum(q * q, axis=1).reshape(NB, BLOCK)
    qbmax = qsq.max(axis=1, keepdims=True)
    kmax = jnp.max(jnp.sum(k * k, axis=1))
    cb = SCALE * jnp.sqrt(qbmax * kmax)
    cbm_ref[0] = cb + jnp.where(keep, 0.0, 1e30)


def _attn_kernel(q_ref, k_ref, v_ref, cbm_ref, o_ref):
    q = q_ref[0]
    cbm = cbm_ref[0, 0]
    l = None
    acc = None
    for c in range(NKC):
        kc = k_ref[0, c * KCHUNK:(c + 1) * KCHUNK, :]
        s = jax.lax.dot_general(q, kc, (((1,), (1,)), ((), ())),
                                preferred_element_type=jnp.float32)
        sub = jnp.repeat(cbm[:, c * KB_PER_CHUNK:(c + 1) * KB_PER_CHUNK],
                         BLOCK, axis=1)
        pf = jnp.exp(s.reshape(QB_PER_TILE, BLOCK, KCHUNK)
                     - sub[:, None, :]).reshape(QTILE, KCHUNK)
        ls = pf.sum(axis=1, keepdims=True)
        vc = v_ref[0, c * KCHUNK:(c + 1) * KCHUNK, :]
        pv = jax.lax.dot_general(pf.astype(jnp.bfloat16), vc,
                                 (((1,), (0,)), ((), ())),
                                 preferred_element_type=jnp.float32)
        if c == 0:
            l = ls
            acc = pv
        else:
            l = l + ls
            acc = acc + pv
    o_ref[0] = acc / l


@functools.partial(jax.jit, static_argnames=("interpret",))
def _run(q3, k3, v3, interpret=False):
    cbm, qb, kb = pl.pallas_call(
        _mask_kernel,
        grid=(H,),
        in_specs=[
            pl.BlockSpec((1, S, D), lambda h: (h, 0, 0)),
            pl.BlockSpec((1, S, D), lambda h: (h, 0, 0)),
        ],
        out_specs=[
            pl.BlockSpec((1, NB, NB), lambda h: (h, 0, 0)),
            pl.BlockSpec((1, S, D), lambda h: (h, 0, 0)),
            pl.BlockSpec((1, S, D), lambda h: (h, 0, 0)),
        ],
        out_shape=[
            jax.ShapeDtypeStruct((H, NB, NB), jnp.float32),
            jax.ShapeDtypeStruct((H, S, D), jnp.bfloat16),
            jax.ShapeDtypeStruct((H, S, D), jnp.bfloat16),
        ],
        interpret=interpret,
    )(q3, k3)

    cbm4 = cbm.reshape(H, NQT, QB_PER_TILE, NB)
    vb = v3.astype(jnp.bfloat16)

    o3 = pl.pallas_call(
        _attn_kernel,
        grid=(H, NQT),
        in_specs=[
            pl.BlockSpec((1, QTILE, D), lambda h, i: (h, i, 0)),
            pl.BlockSpec((1, S, D), lambda h, i: (h, 0, 0)),
            pl.BlockSpec((1, S, D), lambda h, i: (h, 0, 0)),
            pl.BlockSpec((1, 1, QB_PER_TILE, NB), lambda h, i: (h, i, 0, 0)),
        ],
        out_specs=pl.BlockSpec((1, QTILE, D), lambda h, i: (h, i, 0)),
        out_shape=jax.ShapeDtypeStruct((H, S, D), jnp.float32),
        interpret=interpret,
    )(qb, kb, vb, cbm4)
    return o3


def kernel(q, k, v):
    q3 = q[0]
    k3 = k[0]
    v3 = v[0]
    return _run(q3, k3, v3)[None]


# speedup vs baseline: 1.2686x; 1.2686x over previous
import functools

import jax
import jax.numpy as jnp
import numpy as np
from jax.experimental import pallas as pl
from jax.experimental.pallas import tpu as pltpu

BLOCK = 128
NB = 16
H = 16
S = 2048
D = 128
SCALE = 1.0 / np.sqrt(D)
MIN_RETAIN = 1
MAX_RETAIN = 11
ENERGY = 0.95
QTILE = 1024
NQT = S // QTILE
QB_PER_TILE = QTILE // BLOCK
KCHUNK = 256
NKC = S // KCHUNK
KB_PER_CHUNK = KCHUNK // BLOCK


def _mask_kernel(q_ref, k_ref, cbm_ref, qb_ref, kb_ref):
    q = q_ref[0]
    k = k_ref[0]
    qb_ref[0] = (q * SCALE).astype(jnp.bfloat16)
    kb_ref[0] = k.astype(jnp.bfloat16)

    qp = q.reshape(NB, BLOCK, D).mean(axis=1)
    kp = k.reshape(NB, BLOCK, D).mean(axis=1)
    s = jax.lax.dot_general(qp, kp, (((1,), (1,)), ((), ())),
                            preferred_element_type=jnp.float32) * SCALE
    s = s - s.max(axis=-1, keepdims=True)
    e = jnp.exp(s)
    p = e / e.sum(axis=-1, keepdims=True)

    l_idx = jax.lax.broadcasted_iota(jnp.int32, (NB, NB), 1)
    rank_cols = []
    scum_cols = []
    for j in range(NB):
        pj = p[:, j:j + 1]
        before = (p > pj) | ((p == pj) & (l_idx < j))
        rank_cols.append(before.sum(axis=1, keepdims=True).astype(jnp.int32))
        scum_cols.append(pj + jnp.sum(jnp.where(before, p, 0.0), axis=1,
                                      keepdims=True))
    rank = jnp.concatenate(rank_cols, axis=1)
    scum = jnp.concatenate(scum_cols, axis=1)

    theta = ENERGY * p.sum(axis=-1, keepdims=True)
    k_idx = jnp.sum((scum < theta).astype(jnp.int32), axis=1, keepdims=True)
    k_keep = jnp.clip(k_idx, MIN_RETAIN, MAX_RETAIN)
    keep = rank < k_keep

    qsq = jnp.sum(q * q, axis=1).reshape(NB, BLOCK)
    qbmax = qsq.max(axis=1, keepdims=True)
    kmax = jnp.max(jnp.sum(k * k, axis=1))
    cb = SCALE * jnp.sqrt(qbmax * kmax)
    cbm_ref[0] = cb + jnp.where(keep, 0.0, 1e30)


def _attn_kernel(q_ref, k_ref, v_ref, cbm_ref, o_ref):
    q = q_ref[0]
    cbm = cbm_ref[0, 0]
    l = None
    acc = None
    for c in range(NKC):
        kc = k_ref[0, c * KCHUNK:(c + 1) * KCHUNK, :]
        s = jax.lax.dot_general(q, kc, (((1,), (1,)), ((), ())),
                                preferred_element_type=jnp.float32)
        sub = jnp.repeat(cbm[:, c * KB_PER_CHUNK:(c + 1) * KB_PER_CHUNK],
                         BLOCK, axis=1)
        pf = jnp.exp(s.reshape(QB_PER_TILE, BLOCK, KCHUNK)
                     - sub[:, None, :]).reshape(QTILE, KCHUNK)
        ls = pf.sum(axis=1, keepdims=True)
        vc = v_ref[0, c * KCHUNK:(c + 1) * KCHUNK, :]
        pv = jax.lax.dot_general(pf.astype(jnp.bfloat16), vc,
                                 (((1,), (0,)), ((), ())),
                                 preferred_element_type=jnp.float32)
        if c == 0:
            l = ls
            acc = pv
        else:
            l = l + ls
            acc = acc + pv
    o_ref[0] = acc / l


@functools.partial(jax.jit, static_argnames=("interpret",))
def _run(q3, k3, v3, interpret=False):
    cbm, qb, kb = pl.pallas_call(
        _mask_kernel,
        grid=(H,),
        in_specs=[
            pl.BlockSpec((1, S, D), lambda h: (h, 0, 0)),
            pl.BlockSpec((1, S, D), lambda h: (h, 0, 0)),
        ],
        out_specs=[
            pl.BlockSpec((1, NB, NB), lambda h: (h, 0, 0)),
            pl.BlockSpec((1, S, D), lambda h: (h, 0, 0)),
            pl.BlockSpec((1, S, D), lambda h: (h, 0, 0)),
        ],
        out_shape=[
            jax.ShapeDtypeStruct((H, NB, NB), jnp.float32),
            jax.ShapeDtypeStruct((H, S, D), jnp.bfloat16),
            jax.ShapeDtypeStruct((H, S, D), jnp.bfloat16),
        ],
        interpret=interpret,
    )(q3, k3)

    cbm4 = cbm.reshape(H, NQT, QB_PER_TILE, NB)
    vb = v3.astype(jnp.bfloat16)

    o3 = pl.pallas_call(
        _attn_kernel,
        grid=(H, NQT),
        in_specs=[
            pl.BlockSpec((1, QTILE, D), lambda h, i: (h, i, 0)),
            pl.BlockSpec((1, S, D), lambda h, i: (h, 0, 0)),
            pl.BlockSpec((1, S, D), lambda h, i: (h, 0, 0)),
            pl.BlockSpec((1, 1, QB_PER_TILE, NB), lambda h, i: (h, i, 0, 0)),
        ],
        out_specs=pl.BlockSpec((1, QTILE, D), lambda h, i: (h, i, 0)),
        out_shape=jax.ShapeDtypeStruct((H, S, D), jnp.float32),
        interpret=interpret,
    )(qb, kb, vb, cbm4)
    return o3


def kernel(q, k, v):
    q3 = q[0]
    k3 = k[0]
    v3 = v[0]
    return _run(q3, k3, v3)[None]


# speedup vs baseline: 1.3170x; 1.0382x over previous
import functools

import jax
import jax.numpy as jnp
import numpy as np
from jax.experimental import pallas as pl
from jax.experimental.pallas import tpu as pltpu

BLOCK = 128
NB = 16
H = 16
S = 2048
D = 128
SCALE = 1.0 / np.sqrt(D)
MIN_RETAIN = 1
MAX_RETAIN = 11
ENERGY = 0.95
QTILE = 2048
NQT = S // QTILE
QB_PER_TILE = QTILE // BLOCK
KCHUNK = 256
NKC = S // KCHUNK
KB_PER_CHUNK = KCHUNK // BLOCK


def _mask_kernel(q_ref, k_ref, cbm_ref, qb_ref, kb_ref):
    q = q_ref[0]
    k = k_ref[0]
    qb_ref[0] = (q * SCALE).astype(jnp.bfloat16)
    kb_ref[0] = k.astype(jnp.bfloat16)

    qp = q.reshape(NB, BLOCK, D).mean(axis=1)
    kp = k.reshape(NB, BLOCK, D).mean(axis=1)
    s = jax.lax.dot_general(qp, kp, (((1,), (1,)), ((), ())),
                            preferred_element_type=jnp.float32) * SCALE
    s = s - s.max(axis=-1, keepdims=True)
    e = jnp.exp(s)
    p = e / e.sum(axis=-1, keepdims=True)

    l_idx = jax.lax.broadcasted_iota(jnp.int32, (NB, NB), 1)
    rank_cols = []
    scum_cols = []
    for j in range(NB):
        pj = p[:, j:j + 1]
        before = (p > pj) | ((p == pj) & (l_idx < j))
        rank_cols.append(before.sum(axis=1, keepdims=True).astype(jnp.int32))
        scum_cols.append(pj + jnp.sum(jnp.where(before, p, 0.0), axis=1,
                                      keepdims=True))
    rank = jnp.concatenate(rank_cols, axis=1)
    scum = jnp.concatenate(scum_cols, axis=1)

    theta = ENERGY * p.sum(axis=-1, keepdims=True)
    k_idx = jnp.sum((scum < theta).astype(jnp.int32), axis=1, keepdims=True)
    k_keep = jnp.clip(k_idx, MIN_RETAIN, MAX_RETAIN)
    keep = rank < k_keep

    qsq = jnp.sum(q * q, axis=1).reshape(NB, BLOCK)
    qbmax = qsq.max(axis=1, keepdims=True)
    kmax = jnp.max(jnp.sum(k * k, axis=1))
    cb = SCALE * jnp.sqrt(qbmax * kmax)
    cbm_ref[0] = cb + jnp.where(keep, 0.0, 1e30)


def _attn_kernel(q_ref, k_ref, v_ref, cbm_ref, o_ref):
    q = q_ref[0]
    cbm = cbm_ref[0, 0]
    l = None
    acc = None
    for c in range(NKC):
        kc = k_ref[0, c * KCHUNK:(c + 1) * KCHUNK, :]
        s = jax.lax.dot_general(q, kc, (((1,), (1,)), ((), ())),
                                preferred_element_type=jnp.float32)
        sub = jnp.repeat(cbm[:, c * KB_PER_CHUNK:(c + 1) * KB_PER_CHUNK],
                         BLOCK, axis=1)
        pf = jnp.exp(s.reshape(QB_PER_TILE, BLOCK, KCHUNK)
                     - sub[:, None, :]).reshape(QTILE, KCHUNK)
        ls = pf.sum(axis=1, keepdims=True)
        vc = v_ref[0, c * KCHUNK:(c + 1) * KCHUNK, :]
        pv = jax.lax.dot_general(pf.astype(jnp.bfloat16), vc,
                                 (((1,), (0,)), ((), ())),
                                 preferred_element_type=jnp.float32)
        if c == 0:
            l = ls
            acc = pv
        else:
            l = l + ls
            acc = acc + pv
    o_ref[0] = acc / l


@functools.partial(jax.jit, static_argnames=("interpret",))
def _run(q3, k3, v3, interpret=False):
    cbm, qb, kb = pl.pallas_call(
        _mask_kernel,
        grid=(H,),
        in_specs=[
            pl.BlockSpec((1, S, D), lambda h: (h, 0, 0)),
            pl.BlockSpec((1, S, D), lambda h: (h, 0, 0)),
        ],
        out_specs=[
            pl.BlockSpec((1, NB, NB), lambda h: (h, 0, 0)),
            pl.BlockSpec((1, S, D), lambda h: (h, 0, 0)),
            pl.BlockSpec((1, S, D), lambda h: (h, 0, 0)),
        ],
        out_shape=[
            jax.ShapeDtypeStruct((H, NB, NB), jnp.float32),
            jax.ShapeDtypeStruct((H, S, D), jnp.bfloat16),
            jax.ShapeDtypeStruct((H, S, D), jnp.bfloat16),
        ],
        interpret=interpret,
    )(q3, k3)

    cbm4 = cbm.reshape(H, NQT, QB_PER_TILE, NB)
    vb = v3.astype(jnp.bfloat16)

    o3 = pl.pallas_call(
        _attn_kernel,
        grid=(H, NQT),
        in_specs=[
            pl.BlockSpec((1, QTILE, D), lambda h, i: (h, i, 0)),
            pl.BlockSpec((1, S, D), lambda h, i: (h, 0, 0)),
            pl.BlockSpec((1, S, D), lambda h, i: (h, 0, 0)),
            pl.BlockSpec((1, 1, QB_PER_TILE, NB), lambda h, i: (h, i, 0, 0)),
        ],
        out_specs=pl.BlockSpec((1, QTILE, D), lambda h, i: (h, i, 0)),
        out_shape=jax.ShapeDtypeStruct((H, S, D), jnp.float32),
        interpret=interpret,
    )(qb, kb, vb, cbm4)
    return o3


def kernel(q, k, v):
    q3 = q[0]
    k3 = k[0]
    v3 = v[0]
    return _run(q3, k3, v3)[None]
